# baseline (speedup 1.0000x reference)
import functools

import jax
import jax.numpy as jnp
from jax import lax
from jax.experimental import pallas as pl
from jax.experimental.pallas import tpu as pltpu
from jax.experimental.pallas import tpu_sc as plsc

N_TOKENS = 8192
D_MODEL = 4096
C_DIM = 256
N_CODES = 64
BN = 512


def _encode_block(x_ref, wenc_ref, benc_ref, g_ref, b_ref, embt_ref,
                  esq_ref, ze_ref, idx_ref, loss_ref):
    acc = jnp.dot(x_ref[...], wenc_ref[...],
                  preferred_element_type=jnp.float32) + benc_ref[...]
    mu = jnp.mean(acc, axis=-1, keepdims=True)
    var = jnp.mean((acc - mu) ** 2, axis=-1, keepdims=True)
    ze = (acc - mu) / jnp.sqrt(var + 1e-5) * g_ref[...] + b_ref[...]
    ze_ref[...] = ze
    zsq = jnp.sum(ze * ze, axis=-1, keepdims=True)
    cross = jnp.dot(ze, embt_ref[...], preferred_element_type=jnp.float32)
    d = zsq - 2.0 * cross + esq_ref[...]
    dmin = jnp.min(d, axis=1, keepdims=True)
    iota = lax.broadcasted_iota(jnp.int32, d.shape, 1)
    idx = jnp.min(jnp.where(d == dmin, iota, jnp.int32(2**30)), axis=1)
    idx_ref[...] = idx

    @pl.when(pl.program_id(0) == 0)
    def _():
        loss_ref[...] = jnp.zeros_like(loss_ref)

    loss_ref[...] += jnp.sum(dmin, axis=0, keepdims=True)


def _decode_table_block(emb_ref, wdec_ref, bdec_ref, out_ref):
    out_ref[...] = jnp.dot(emb_ref[...], wdec_ref[...],
                           preferred_element_type=jnp.float32) + bdec_ref[...]


def _recon_block(idx_ref, dec_ref, xr_ref):
    idx = idx_ref[...]
    onehot = (lax.broadcasted_iota(jnp.int32, (BN, N_CODES), 1)
              == idx[:, None]).astype(jnp.float32)
    xr_ref[...] = jnp.dot(onehot, dec_ref[...],
                          preferred_element_type=jnp.float32)


def _sc_info():
    try:
        info = plsc.get_sparse_core_info()
        return info.num_cores, info.num_subcores
    except Exception:
        return 2, 16


def _zq_compose_body(emb_hbm, idx_hbm, zq_hbm, table_v, idx_v, zq_v,
                     sem_o, *, n_cores, b_per_w):
    wid = lax.axis_index("s") * n_cores + lax.axis_index("c")
    base = wid * b_per_w
    pltpu.sync_copy(emb_hbm, table_v)
    pltpu.sync_copy(idx_hbm.at[pl.ds(base, b_per_w)], idx_v)
    lane = lax.broadcasted_iota(jnp.int32, (16,), 0)
    n_groups = b_per_w // 16
    half_words = (b_per_w // 2) * C_DIM
    descs = []
    for h in range(2):
        g0 = h * (n_groups // 2)
        rowbases = [idx_v[pl.ds((g0 + g) * 16, 16)] * C_DIM
                    for g in range(n_groups // 2)]
        outbases = [(lane + (g0 + g) * 16) * C_DIM
                    for g in range(n_groups // 2)]

        @plsc.parallel_loop(0, C_DIM, unroll=2)
        def col(c):
            for rb, ob in zip(rowbases, outbases):
                vals = plsc.load_gather(table_v, [rb + c])
                plsc.store_scatter(zq_v, [ob + c], vals)

        descs.append(pltpu.make_async_copy(
            zq_v.at[pl.ds(h * half_words, half_words)],
            zq_hbm.at[pl.ds(base * C_DIM + h * half_words, half_words)],
            sem_o))
        descs[-1].start()
    for d in descs:
        d.wait()


def kernel(x, modality, W_enc, b_enc, ln_g, ln_b, embeddings, W_dec, b_dec):
    del modality
    esq = jnp.sum(embeddings * embeddings, axis=-1).reshape(1, N_CODES)
    embt = embeddings.T

    n_blocks = N_TOKENS // BN
    ze, idx, loss_sum = pl.pallas_call(
        _encode_block,
        grid=(n_blocks,),
        in_specs=[
            pl.BlockSpec((BN, D_MODEL), lambda i: (i, 0)),
            pl.BlockSpec((D_MODEL, C_DIM), lambda i: (0, 0)),
            pl.BlockSpec((1, C_DIM), lambda i: (0, 0)),
            pl.BlockSpec((1, C_DIM), lambda i: (0, 0)),
            pl.BlockSpec((1, C_DIM), lambda i: (0, 0)),
            pl.BlockSpec((C_DIM, N_CODES), lambda i: (0, 0)),
            pl.BlockSpec((1, N_CODES), lambda i: (0, 0)),
        ],
        out_specs=[
            pl.BlockSpec((BN, C_DIM), lambda i: (i, 0)),
            pl.BlockSpec((BN,), lambda i: (i,)),
            pl.BlockSpec((1, 1), lambda i: (0, 0)),
        ],
        out_shape=[
            jax.ShapeDtypeStruct((N_TOKENS, C_DIM), jnp.float32),
            jax.ShapeDtypeStruct((N_TOKENS,), jnp.int32),
            jax.ShapeDtypeStruct((1, 1), jnp.float32),
        ],
        compiler_params=pltpu.CompilerParams(
            dimension_semantics=("arbitrary",)),
    )(x, W_enc, b_enc.reshape(1, C_DIM), ln_g.reshape(1, C_DIM),
      ln_b.reshape(1, C_DIM), embt, esq)

    decoded = pl.pallas_call(
        _decode_table_block,
        out_shape=jax.ShapeDtypeStruct((N_CODES, D_MODEL), jnp.float32),
    )(embeddings, W_dec, b_dec.reshape(1, D_MODEL))

    def recon_call(idx_part, n_tok):
        return pl.pallas_call(
            _recon_block,
            grid=(n_tok // BN,),
            in_specs=[
                pl.BlockSpec((BN,), lambda i: (i,)),
                pl.BlockSpec((N_CODES, D_MODEL), lambda i: (0, 0)),
            ],
            out_specs=pl.BlockSpec((BN, D_MODEL), lambda i: (i, 0)),
            out_shape=jax.ShapeDtypeStruct((n_tok, D_MODEL), jnp.float32),
            compiler_params=pltpu.CompilerParams(
                dimension_semantics=("arbitrary",)),
        )(idx_part, decoded)

    half = N_TOKENS // 2
    x_recon_a = recon_call(idx[:half], half)

    nc, ns = _sc_info()
    b_per_w = N_TOKENS // (nc * ns)
    mesh = plsc.VectorSubcoreMesh(core_axis_name="c", subcore_axis_name="s")
    z_q = pl.kernel(
        functools.partial(_zq_compose_body, n_cores=nc, b_per_w=b_per_w),
        out_type=jax.ShapeDtypeStruct((N_TOKENS * C_DIM,), jnp.float32),
        mesh=mesh,
        scratch_types=[
            pltpu.VMEM((N_CODES * C_DIM,), jnp.float32),
            pltpu.VMEM((b_per_w,), jnp.int32),
            pltpu.VMEM((b_per_w * C_DIM,), jnp.float32),
            pltpu.SemaphoreType.DMA,
        ],
        compiler_params=pltpu.CompilerParams(needs_layout_passes=False),
    )(embeddings.reshape(N_CODES * C_DIM), idx)
    z_q = z_q.reshape(N_TOKENS, C_DIM)

    x_recon_b = recon_call(idx[half:], half)
    x_recon = jnp.concatenate([x_recon_a, x_recon_b], axis=0)

    loss = (loss_sum[0, 0] / (N_TOKENS * C_DIM)).reshape(())
    return (x_recon, loss, idx, ze, z_q)


# speedup vs baseline: 1.6306x; 1.6306x over previous
import functools

import jax
import jax.numpy as jnp
from jax import lax
from jax.experimental import pallas as pl
from jax.experimental.pallas import tpu as pltpu
from jax.experimental.pallas import tpu_sc as plsc

N_TOKENS = 8192
D_MODEL = 4096
C_DIM = 256
N_CODES = 64
BN = 512


def _encode_block(x_ref, wenc_ref, benc_ref, g_ref, b_ref, embt_ref,
                  esq_ref, ze_ref, idx_ref, loss_ref):
    acc = jnp.dot(x_ref[...], wenc_ref[...],
                  preferred_element_type=jnp.float32) + benc_ref[...]
    mu = jnp.mean(acc, axis=-1, keepdims=True)
    var = jnp.mean((acc - mu) ** 2, axis=-1, keepdims=True)
    ze = (acc - mu) / jnp.sqrt(var + 1e-5) * g_ref[...] + b_ref[...]
    ze_ref[...] = ze
    zsq = jnp.sum(ze * ze, axis=-1, keepdims=True)
    cross = jnp.dot(ze, embt_ref[...], preferred_element_type=jnp.float32)
    d = zsq - 2.0 * cross + esq_ref[...]
    dmin = jnp.min(d, axis=1, keepdims=True)
    iota = lax.broadcasted_iota(jnp.int32, d.shape, 1)
    idx = jnp.min(jnp.where(d == dmin, iota, jnp.int32(2**30)), axis=1)
    idx_ref[...] = idx

    @pl.when(pl.program_id(0) == 0)
    def _():
        loss_ref[...] = jnp.zeros_like(loss_ref)

    loss_ref[...] += jnp.sum(dmin, axis=0, keepdims=True)


def _decode_table_block(emb_ref, wdec_ref, bdec_ref, out_ref):
    out_ref[...] = jnp.dot(emb_ref[...], wdec_ref[...],
                           preferred_element_type=jnp.float32) + bdec_ref[...]


def _recon_block(idx_ref, dec_ref, xr_ref):
    idx = idx_ref[...]
    onehot = (lax.broadcasted_iota(jnp.int32, (BN, N_CODES), 1)
              == idx[:, None]).astype(jnp.float32)
    xr_ref[...] = jnp.dot(onehot, dec_ref[...],
                          preferred_element_type=jnp.float32)


def _sc_info():
    try:
        info = plsc.get_sparse_core_info()
        return info.num_cores, info.num_subcores
    except Exception:
        return 2, 16


def _zq_compose_body(emb_hbm, idx_hbm, zq_hbm, table_v, idx_v, zq_v,
                     sem_o, *, n_cores, b_per_w):
    wid = lax.axis_index("s") * n_cores + lax.axis_index("c")
    base = wid * b_per_w
    pltpu.sync_copy(emb_hbm, table_v)
    pltpu.sync_copy(idx_hbm.at[pl.ds(base, b_per_w)], idx_v)
    lane = lax.broadcasted_iota(jnp.int32, (16,), 0)
    n_groups = b_per_w // 16
    half_words = (b_per_w // 2) * C_DIM
    descs = []
    for h in range(2):
        g0 = h * (n_groups // 2)
        rowbases = [idx_v[pl.ds((g0 + g) * 16, 16)] * C_DIM
                    for g in range(n_groups // 2)]
        outbases = [(lane + (g0 + g) * 16) * C_DIM
                    for g in range(n_groups // 2)]

        @plsc.parallel_loop(0, C_DIM, unroll=2)
        def col(c):
            for rb, ob in zip(rowbases, outbases):
                vals = plsc.load_gather(table_v, [rb + c])
                plsc.store_scatter(zq_v, [ob + c], vals)

        descs.append(pltpu.make_async_copy(
            zq_v.at[pl.ds(h * half_words, half_words)],
            zq_hbm.at[pl.ds(base * C_DIM + h * half_words, half_words)],
            sem_o))
        descs[-1].start()
    for d in descs:
        d.wait()


def kernel(x, modality, W_enc, b_enc, ln_g, ln_b, embeddings, W_dec, b_dec):
    del modality
    esq = jnp.sum(embeddings * embeddings, axis=-1).reshape(1, N_CODES)
    embt = embeddings.T

    n_blocks = N_TOKENS // BN
    ze, idx, loss_sum = pl.pallas_call(
        _encode_block,
        grid=(n_blocks,),
        in_specs=[
            pl.BlockSpec((BN, D_MODEL), lambda i: (i, 0)),
            pl.BlockSpec((D_MODEL, C_DIM), lambda i: (0, 0)),
            pl.BlockSpec((1, C_DIM), lambda i: (0, 0)),
            pl.BlockSpec((1, C_DIM), lambda i: (0, 0)),
            pl.BlockSpec((1, C_DIM), lambda i: (0, 0)),
            pl.BlockSpec((C_DIM, N_CODES), lambda i: (0, 0)),
            pl.BlockSpec((1, N_CODES), lambda i: (0, 0)),
        ],
        out_specs=[
            pl.BlockSpec((BN, C_DIM), lambda i: (i, 0)),
            pl.BlockSpec((BN,), lambda i: (i,)),
            pl.BlockSpec((1, 1), lambda i: (0, 0)),
        ],
        out_shape=[
            jax.ShapeDtypeStruct((N_TOKENS, C_DIM), jnp.float32),
            jax.ShapeDtypeStruct((N_TOKENS,), jnp.int32),
            jax.ShapeDtypeStruct((1, 1), jnp.float32),
        ],
        compiler_params=pltpu.CompilerParams(
            dimension_semantics=("arbitrary",)),
    )(x, W_enc, b_enc.reshape(1, C_DIM), ln_g.reshape(1, C_DIM),
      ln_b.reshape(1, C_DIM), embt, esq)

    decoded = pl.pallas_call(
        _decode_table_block,
        out_shape=jax.ShapeDtypeStruct((N_CODES, D_MODEL), jnp.float32),
    )(embeddings, W_dec, b_dec.reshape(1, D_MODEL))

    x_recon = pl.pallas_call(
        _recon_block,
        grid=(n_blocks,),
        in_specs=[
            pl.BlockSpec((BN,), lambda i: (i,)),
            pl.BlockSpec((N_CODES, D_MODEL), lambda i: (0, 0)),
        ],
        out_specs=pl.BlockSpec((BN, D_MODEL), lambda i: (i, 0)),
        out_shape=jax.ShapeDtypeStruct((N_TOKENS, D_MODEL), jnp.float32),
        compiler_params=pltpu.CompilerParams(
            dimension_semantics=("arbitrary",)),
    )(idx, decoded)

    nc, ns = _sc_info()
    b_per_w = N_TOKENS // (nc * ns)
    mesh = plsc.VectorSubcoreMesh(core_axis_name="c", subcore_axis_name="s")
    z_q = pl.kernel(
        functools.partial(_zq_compose_body, n_cores=nc, b_per_w=b_per_w),
        out_type=jax.ShapeDtypeStruct((N_TOKENS * C_DIM,), jnp.float32),
        mesh=mesh,
        scratch_types=[
            pltpu.VMEM((N_CODES * C_DIM,), jnp.float32),
            pltpu.VMEM((b_per_w,), jnp.int32),
            pltpu.VMEM((b_per_w * C_DIM,), jnp.float32),
            pltpu.SemaphoreType.DMA,
        ],
        compiler_params=pltpu.CompilerParams(needs_layout_passes=False),
    )(embeddings.reshape(N_CODES * C_DIM), idx)
    z_q = z_q.reshape(N_TOKENS, C_DIM)

    loss = (loss_sum[0, 0] / (N_TOKENS * C_DIM)).reshape(())
    return (x_recon, loss, idx, ze, z_q)


# speedup vs baseline: 1.6321x; 1.0009x over previous
import functools

import jax
import jax.numpy as jnp
from jax import lax
from jax.experimental import pallas as pl
from jax.experimental.pallas import tpu as pltpu
from jax.experimental.pallas import tpu_sc as plsc

N_TOKENS = 8192
D_MODEL = 4096
C_DIM = 256
N_CODES = 64
BN = 512


def _encode_block(x_ref, wenc_ref, benc_ref, g_ref, b_ref, embt_ref,
                  esq_ref, ze_ref, idx_ref, loss_ref):
    acc = jnp.dot(x_ref[...], wenc_ref[...],
                  preferred_element_type=jnp.float32) + benc_ref[...]
    mu = jnp.mean(acc, axis=-1, keepdims=True)
    var = jnp.mean((acc - mu) ** 2, axis=-1, keepdims=True)
    ze = (acc - mu) / jnp.sqrt(var + 1e-5) * g_ref[...] + b_ref[...]
    ze_ref[...] = ze
    zsq = jnp.sum(ze * ze, axis=-1, keepdims=True)
    cross = jnp.dot(ze, embt_ref[...], preferred_element_type=jnp.float32)
    d = zsq - 2.0 * cross + esq_ref[...]
    dmin = jnp.min(d, axis=1, keepdims=True)
    iota = lax.broadcasted_iota(jnp.int32, d.shape, 1)
    idx = jnp.min(jnp.where(d == dmin, iota, jnp.int32(2**30)), axis=1)
    idx_ref[...] = idx

    @pl.when(pl.program_id(0) == 0)
    def _():
        loss_ref[...] = jnp.zeros_like(loss_ref)

    loss_ref[...] += jnp.sum(dmin, axis=0, keepdims=True)


def _decode_table_block(emb_ref, wdec_ref, bdec_ref, out_ref):
    out_ref[...] = jnp.dot(emb_ref[...], wdec_ref[...],
                           preferred_element_type=jnp.float32) + bdec_ref[...]


def _recon_block(idx_ref, dec_ref, xr_ref):
    idx = idx_ref[...]
    onehot = (lax.broadcasted_iota(jnp.int32, (BN, N_CODES), 1)
              == idx[:, None]).astype(jnp.float32)
    xr_ref[...] = jnp.dot(onehot, dec_ref[...],
                          preferred_element_type=jnp.float32)


def _sc_info():
    try:
        info = plsc.get_sparse_core_info()
        return info.num_cores, info.num_subcores
    except Exception:
        return 2, 16


def _zq_compose_body(emb_hbm, idx_hbm, zq_hbm, table_v, idx_v, zq_v,
                     sem_o, *, n_cores, b_per_w):
    wid = lax.axis_index("s") * n_cores + lax.axis_index("c")
    base = wid * b_per_w
    pltpu.sync_copy(emb_hbm, table_v)
    pltpu.sync_copy(idx_hbm.at[pl.ds(base, b_per_w)], idx_v)
    lane = lax.broadcasted_iota(jnp.int32, (16,), 0)
    n_groups = b_per_w // 16
    half_words = (b_per_w // 2) * C_DIM
    descs = []
    for h in range(2):
        g0 = h * (n_groups // 2)
        rowbases = [idx_v[pl.ds((g0 + g) * 16, 16)] * C_DIM
                    for g in range(n_groups // 2)]
        outbases = [(lane + (g0 + g) * 16) * C_DIM
                    for g in range(n_groups // 2)]

        @plsc.parallel_loop(0, C_DIM, unroll=4)
        def col(c):
            for rb, ob in zip(rowbases, outbases):
                vals = plsc.load_gather(table_v, [rb + c])
                plsc.store_scatter(zq_v, [ob + c], vals)

        descs.append(pltpu.make_async_copy(
            zq_v.at[pl.ds(h * half_words, half_words)],
            zq_hbm.at[pl.ds(base * C_DIM + h * half_words, half_words)],
            sem_o))
        descs[-1].start()
    for d in descs:
        d.wait()


def kernel(x, modality, W_enc, b_enc, ln_g, ln_b, embeddings, W_dec, b_dec):
    del modality
    esq = jnp.sum(embeddings * embeddings, axis=-1).reshape(1, N_CODES)
    embt = embeddings.T

    n_blocks = N_TOKENS // BN
    ze, idx, loss_sum = pl.pallas_call(
        _encode_block,
        grid=(n_blocks,),
        in_specs=[
            pl.BlockSpec((BN, D_MODEL), lambda i: (i, 0)),
            pl.BlockSpec((D_MODEL, C_DIM), lambda i: (0, 0)),
            pl.BlockSpec((1, C_DIM), lambda i: (0, 0)),
            pl.BlockSpec((1, C_DIM), lambda i: (0, 0)),
            pl.BlockSpec((1, C_DIM), lambda i: (0, 0)),
            pl.BlockSpec((C_DIM, N_CODES), lambda i: (0, 0)),
            pl.BlockSpec((1, N_CODES), lambda i: (0, 0)),
        ],
        out_specs=[
            pl.BlockSpec((BN, C_DIM), lambda i: (i, 0)),
            pl.BlockSpec((BN,), lambda i: (i,)),
            pl.BlockSpec((1, 1), lambda i: (0, 0)),
        ],
        out_shape=[
            jax.ShapeDtypeStruct((N_TOKENS, C_DIM), jnp.float32),
            jax.ShapeDtypeStruct((N_TOKENS,), jnp.int32),
            jax.ShapeDtypeStruct((1, 1), jnp.float32),
        ],
        compiler_params=pltpu.CompilerParams(
            dimension_semantics=("arbitrary",)),
    )(x, W_enc, b_enc.reshape(1, C_DIM), ln_g.reshape(1, C_DIM),
      ln_b.reshape(1, C_DIM), embt, esq)

    decoded = pl.pallas_call(
        _decode_table_block,
        out_shape=jax.ShapeDtypeStruct((N_CODES, D_MODEL), jnp.float32),
    )(embeddings, W_dec, b_dec.reshape(1, D_MODEL))

    x_recon = pl.pallas_call(
        _recon_block,
        grid=(n_blocks,),
        in_specs=[
            pl.BlockSpec((BN,), lambda i: (i,)),
            pl.BlockSpec((N_CODES, D_MODEL), lambda i: (0, 0)),
        ],
        out_specs=pl.BlockSpec((BN, D_MODEL), lambda i: (i, 0)),
        out_shape=jax.ShapeDtypeStruct((N_TOKENS, D_MODEL), jnp.float32),
        compiler_params=pltpu.CompilerParams(
            dimension_semantics=("arbitrary",)),
    )(idx, decoded)

    nc, ns = _sc_info()
    b_per_w = N_TOKENS // (nc * ns)
    mesh = plsc.VectorSubcoreMesh(core_axis_name="c", subcore_axis_name="s")
    z_q = pl.kernel(
        functools.partial(_zq_compose_body, n_cores=nc, b_per_w=b_per_w),
        out_type=jax.ShapeDtypeStruct((N_TOKENS * C_DIM,), jnp.float32),
        mesh=mesh,
        scratch_types=[
            pltpu.VMEM((N_CODES * C_DIM,), jnp.float32),
            pltpu.VMEM((b_per_w,), jnp.int32),
            pltpu.VMEM((b_per_w * C_DIM,), jnp.float32),
            pltpu.SemaphoreType.DMA,
        ],
        compiler_params=pltpu.CompilerParams(needs_layout_passes=False),
    )(embeddings.reshape(N_CODES * C_DIM), idx)
    z_q = z_q.reshape(N_TOKENS, C_DIM)

    loss = (loss_sum[0, 0] / (N_TOKENS * C_DIM)).reshape(())
    return (x_recon, loss, idx, ze, z_q)
